# baseline (speedup 1.0000x reference)
import functools

import jax
import jax.numpy as jnp
from jax import lax
from jax.experimental import pallas as pl
from jax.experimental.pallas import tpu as pltpu
from jax.experimental.pallas import tpu_sc as plsc

_G = 8
_K = 125


def _blk_dot(adj3, t_ref, col_base, h):
    acc = jnp.zeros((adj3.shape[0], h), jnp.float32)
    for g in range(_G):
        acc += jnp.dot(adj3[:, g, :], t_ref[pl.ds(col_base + g * _K, _K), :],
                       preferred_element_type=jnp.float32)
    return acc


def _prep_body(x_ref, wo_ref, ws_ref, so_ref, ss_ref):
    xb = x_ref[...]
    so_ref[...] = jnp.dot(xb, wo_ref[...], preferred_element_type=jnp.float32)
    ss_ref[...] = jnp.dot(xb, ws_ref[...], preferred_element_type=jnp.float32)


def _pass1_body(bm,
                adj_ref, s_ref, b1_ref, w2_ref,
                t_out, u1_out,
                h_acc, uacc, t_scr, d_scr):
    i = pl.program_id(0)
    j = pl.program_id(1)
    nb = pl.num_programs(1)
    adj = adj_ref[...]
    col = j * bm

    prod = _blk_dot(adj, s_ref, col, s_ref.shape[1])

    @pl.when(j == 0)
    def _():
        h_acc[...] = prod
        uacc[...] = jnp.zeros_like(uacc)

    @pl.when(j > 0)
    def _():
        h_acc[...] += prod

    @pl.when(j < i)
    def _():
        uacc[...] += _blk_dot(adj, t_scr, col, uacc.shape[1])

    @pl.when(j == i)
    def _():
        d_scr[...] = adj

    @pl.when(j == nb - 1)
    def _():
        t = jnp.dot(jnp.maximum(h_acc[...] + b1_ref[...], 0.0),
                    w2_ref[...], preferred_element_type=jnp.float32)
        t_scr[pl.ds(i * bm, bm), :] = t
        t_out[...] = t
        u1_out[...] = uacc[...] + _blk_dot(d_scr[...], t_scr, i * bm,
                                           uacc.shape[1])


def _pass2o_body(bm, adj_ref, t_ref, u1_ref, b2_ref, u_out, uacc):
    i = pl.program_id(0)
    j = pl.program_id(1)
    nb = pl.num_programs(1)

    @pl.when(j == 0)
    def _():
        uacc[...] = jnp.zeros_like(uacc)

    @pl.when(j > i)
    def _():
        uacc[...] += _blk_dot(adj_ref[...], t_ref, j * bm, uacc.shape[1])

    @pl.when(j == nb - 1)
    def _():
        u_out[...] = u1_ref[...] + uacc[...] + b2_ref[...]


def _pass2s_body(bm, adj_ref, t_ref, v1_ref, u_ref,
                 bs2_ref, ag1_ref, ag2_ref, wa_ref, wb_ref, c_ref,
                 pq_ref, vacc):
    i = pl.program_id(0)
    j = pl.program_id(1)
    nb = pl.num_programs(1)

    @pl.when(j == 0)
    def _():
        vacc[...] = jnp.zeros_like(vacc)

    @pl.when(j > i)
    def _():
        vacc[...] += _blk_dot(adj_ref[...], t_ref, j * bm, vacc.shape[1])

    @pl.when(j == nb - 1)
    def _():
        u = u_ref[...]
        v = v1_ref[...] + vacc[...] + bs2_ref[...]
        a_o = jnp.sum(u * ag1_ref[...], axis=1, keepdims=True)
        a_s = jnp.sum(v * ag2_ref[...], axis=1, keepdims=True)
        p = a_o * jnp.sum(u * wa_ref[...], axis=1, keepdims=True) \
            + a_s * jnp.sum(v * wa_ref[...], axis=1, keepdims=True) \
            + c_ref[...]
        q = a_o * jnp.sum(u * wb_ref[...], axis=1, keepdims=True) \
            + a_s * jnp.sum(v * wb_ref[...], axis=1, keepdims=True)
        pq_ref[...] = jnp.concatenate(
            [p, q, jnp.zeros((p.shape[0], 6), jnp.float32)], axis=1)


def _edge_gather(p, q, idx0, idx1, n_workers, nc):
    (e_total,) = idx0.shape
    n_nodes = p.shape[0]
    ew = e_total // n_workers
    steps = (ew + 15) // 16
    pad = steps * 16

    mesh = plsc.VectorSubcoreMesh(core_axis_name="c", subcore_axis_name="s")

    @functools.partial(
        pl.kernel,
        mesh=mesh,
        compiler_params=pltpu.CompilerParams(needs_layout_passes=False),
        out_type=jax.ShapeDtypeStruct((e_total,), jnp.float32),
        scratch_types=[
            pltpu.VMEM((n_nodes,), jnp.float32),
            pltpu.VMEM((n_nodes,), jnp.float32),
            pltpu.VMEM((pad,), jnp.int32),
            pltpu.VMEM((pad,), jnp.int32),
            pltpu.VMEM((pad,), jnp.float32),
        ],
    )
    def k(p_hbm, q_hbm, i0_hbm, i1_hbm, out_hbm, p_v, q_v, i0_v, i1_v, out_v):
        wid = lax.axis_index("s") * nc + lax.axis_index("c")
        base = wid * ew
        pltpu.sync_copy(p_hbm, p_v)
        pltpu.sync_copy(q_hbm, q_v)
        if pad > ew:
            zeros16 = jnp.zeros((16,), jnp.int32)
            i0_v[pl.ds(pad - 16, 16)] = zeros16
            i1_v[pl.ds(pad - 16, 16)] = zeros16
        pltpu.sync_copy(i0_hbm.at[pl.ds(base, ew)], i0_v.at[pl.ds(0, ew)])
        pltpu.sync_copy(i1_hbm.at[pl.ds(base, ew)], i1_v.at[pl.ds(0, ew)])

        def body(k_it, _):
            off = k_it * 16
            g0 = plsc.load_gather(p_v, [i0_v[pl.ds(off, 16)]])
            g1 = plsc.load_gather(q_v, [i1_v[pl.ds(off, 16)]])
            out_v[pl.ds(off, 16)] = g0 + g1
            return _

        lax.fori_loop(0, steps, body, None)
        pltpu.sync_copy(out_v.at[pl.ds(0, ew)], out_hbm.at[pl.ds(base, ew)])

    return k(p, q, idx0, idx1)


def kernel(x, o_adj, s_adj, idx, Wo1, bo1, Wo2, bo2, Ws1, bs1, Ws2, bs2,
           ag1, ag2, Wd1, bd1, Wd2, bd2):
    n, nfeat = x.shape
    h1 = Wo1.shape[1]
    h2 = Wo2.shape[1]

    w = Wd1 @ Wd2
    wa = w[:h2, 0][None, :]
    wb = w[h2:, 0][None, :]
    c = (bd1 @ Wd2 + bd2).reshape(1, 1)

    bmp = 2000
    s_o, s_s = pl.pallas_call(
        _prep_body,
        grid=(pl.cdiv(n, bmp),),
        in_specs=[
            pl.BlockSpec((bmp, nfeat), lambda i: (i, 0)),
            pl.BlockSpec((nfeat, h1), lambda i: (0, 0)),
            pl.BlockSpec((nfeat, h1), lambda i: (0, 0)),
        ],
        out_specs=[
            pl.BlockSpec((bmp, h1), lambda i: (i, 0)),
            pl.BlockSpec((bmp, h1), lambda i: (i, 0)),
        ],
        out_shape=[
            jax.ShapeDtypeStruct((n, h1), jnp.float32),
            jax.ShapeDtypeStruct((n, h1), jnp.float32),
        ],
    )(x, Wo1, Ws1)

    bm = 1000
    nb = n // bm
    o3 = o_adj.reshape(n, nb * _G, _K)
    s3 = s_adj.reshape(n, nb * _G, _K)
    grid2d = (nb, nb)
    adj_ij = pl.BlockSpec((bm, _G, _K), lambda i, j: (i, j, 0))
    row_spec = lambda w_: pl.BlockSpec((bm, w_), lambda i, j: (i, 0))
    res = lambda r, c_: pl.BlockSpec((r, c_), lambda i, j: (0, 0))

    def pass1(adj3, s, b1, w2):
        return pl.pallas_call(
            functools.partial(_pass1_body, bm),
            grid=grid2d,
            in_specs=[adj_ij, res(n, h1), res(1, h1), res(h1, h2)],
            out_specs=[row_spec(h2), row_spec(h2)],
            out_shape=[jax.ShapeDtypeStruct((n, h2), jnp.float32)] * 2,
            scratch_shapes=[
                pltpu.VMEM((bm, h1), jnp.float32),
                pltpu.VMEM((bm, h2), jnp.float32),
                pltpu.VMEM((n, h2), jnp.float32),
                pltpu.VMEM((bm, _G, _K), jnp.float32),
            ],
        )(adj3, s, b1[None, :], w2)

    t_o, u1 = pass1(o3, s_o, bo1, Wo2)
    t_s, v1 = pass1(s3, s_s, bs1, Ws2)

    def adj_upper(i, j):
        return (i, jnp.maximum(j, jnp.minimum(i + 1, nb - 1)), 0)

    adj_up = pl.BlockSpec((bm, _G, _K), adj_upper)

    u = pl.pallas_call(
        functools.partial(_pass2o_body, bm),
        grid=grid2d,
        in_specs=[adj_up, res(n, h2), row_spec(h2), res(1, h2)],
        out_specs=row_spec(h2),
        out_shape=jax.ShapeDtypeStruct((n, h2), jnp.float32),
        scratch_shapes=[pltpu.VMEM((bm, h2), jnp.float32)],
    )(o3, t_o, u1, bo2[None, :])

    pq = pl.pallas_call(
        functools.partial(_pass2s_body, bm),
        grid=grid2d,
        in_specs=[
            adj_up, res(n, h2), row_spec(h2), row_spec(h2),
            res(1, h2), res(1, h2), res(1, h2),
            res(1, h2), res(1, h2), res(1, 1),
        ],
        out_specs=pl.BlockSpec((bm, 8), lambda i, j: (i, 0)),
        out_shape=jax.ShapeDtypeStruct((n, 8), jnp.float32),
        scratch_shapes=[pltpu.VMEM((bm, h2), jnp.float32)],
    )(s3, t_s, v1, u, bs2[None, :],
      ag1[None, :], ag2[None, :], wa, wb, c)

    p = pq[:, 0]
    q = pq[:, 1]

    info = plsc.get_sparse_core_info()
    nc, ns = info.num_cores, info.num_subcores
    out = _edge_gather(p, q, idx[0], idx[1], nc * ns, nc)
    return out[:, None]


# speedup vs baseline: 3.1308x; 3.1308x over previous
import functools

import jax
import jax.numpy as jnp
from jax import lax
from jax.experimental import pallas as pl
from jax.experimental.pallas import tpu as pltpu
from jax.experimental.pallas import tpu_sc as plsc

_BM = 1000
_BN = 1280


def _dot(a, b):
    return jnp.dot(a, b, preferred_element_type=jnp.float32)


def _edge_dot(n, adj, t_ref):
    nbc = pl.cdiv(n, _BN)
    base = (nbc - 1) * _BN
    tail = n - base
    ta = (tail // 128) * 128
    acc = _dot(adj[:, :ta], t_ref[pl.ds(base, ta), :])
    if tail > ta:
        acc += _dot(adj[:, ta:tail], t_ref[pl.ds(base + ta, tail - ta), :])
    return acc


def _prep_body(x_ref, wo_ref, ws_ref, so_ref, ss_ref):
    xb = x_ref[...]
    so_ref[...] = _dot(xb, wo_ref[...])
    ss_ref[...] = _dot(xb, ws_ref[...])


def _pass1_body(n, adj_ref, s_ref, b1_ref, w2_ref, t_out, u1_out,
                h_acc, uacc, t_scr):
    i = pl.program_id(0)
    j = pl.program_id(1)
    nbc = pl.num_programs(1)
    adj = adj_ref[...]
    cov = _BN * (j + 1)

    @pl.when(j == 0)
    def _():
        h_acc[...] = _dot(adj, s_ref[pl.ds(0, _BN), :])
        uacc[...] = jnp.zeros_like(uacc)

    @pl.when((j > 0) & (j < nbc - 1))
    def _():
        h_acc[...] += _dot(adj, s_ref[pl.ds(j * _BN, _BN), :])

    @pl.when(cov <= _BM * i)
    def _():
        uacc[...] += _dot(adj, t_scr[pl.ds(j * _BN, _BN), :])

    @pl.when(j == nbc - 1)
    def _():
        h = h_acc[...] + _edge_dot(n, adj, s_ref)
        t = _dot(jnp.maximum(h + b1_ref[...], 0.0), w2_ref[...])
        t_scr[pl.ds(i * _BM, _BM), :] = t
        t_out[...] = t
        u1_out[...] = uacc[...]


def _pass2o_body(n, adj_ref, t_ref, u1_ref, b2_ref, u_out, uacc):
    i = pl.program_id(0)
    j = pl.program_id(1)
    nbc = pl.num_programs(1)
    cov = jnp.minimum(_BN * (j + 1), n)

    @pl.when(j == 0)
    def _():
        uacc[...] = jnp.zeros_like(uacc)

    @pl.when((cov > _BM * i) & (j < nbc - 1))
    def _():
        uacc[...] += _dot(adj_ref[...], t_ref[pl.ds(j * _BN, _BN), :])

    @pl.when(j == nbc - 1)
    def _():
        u_out[...] = u1_ref[...] + uacc[...] \
            + _edge_dot(n, adj_ref[...], t_ref) + b2_ref[...]


def _pass2s_body(n, adj_ref, t_ref, v1_ref, u_ref,
                 bs2_ref, ag1_ref, ag2_ref, wa_ref, wb_ref, c_ref,
                 pq_ref, vacc):
    i = pl.program_id(0)
    j = pl.program_id(1)
    nbc = pl.num_programs(1)
    cov = jnp.minimum(_BN * (j + 1), n)

    @pl.when(j == 0)
    def _():
        vacc[...] = jnp.zeros_like(vacc)

    @pl.when((cov > _BM * i) & (j < nbc - 1))
    def _():
        vacc[...] += _dot(adj_ref[...], t_ref[pl.ds(j * _BN, _BN), :])

    @pl.when(j == nbc - 1)
    def _():
        u = u_ref[...]
        v = v1_ref[...] + vacc[...] + _edge_dot(n, adj_ref[...], t_ref) \
            + bs2_ref[...]
        a_o = jnp.sum(u * ag1_ref[...], axis=1, keepdims=True)
        a_s = jnp.sum(v * ag2_ref[...], axis=1, keepdims=True)
        p = a_o * jnp.sum(u * wa_ref[...], axis=1, keepdims=True) \
            + a_s * jnp.sum(v * wa_ref[...], axis=1, keepdims=True) \
            + c_ref[...]
        q = a_o * jnp.sum(u * wb_ref[...], axis=1, keepdims=True) \
            + a_s * jnp.sum(v * wb_ref[...], axis=1, keepdims=True)
        pq_ref[...] = jnp.concatenate(
            [p, q, jnp.zeros((p.shape[0], 6), jnp.float32)], axis=1)


def _edge_gather(p, q, idx0, idx1, n_workers, nc):
    (e_total,) = idx0.shape
    n_nodes = p.shape[0]
    ew = e_total // n_workers
    steps = (ew + 15) // 16
    pad = steps * 16

    mesh = plsc.VectorSubcoreMesh(core_axis_name="c", subcore_axis_name="s")

    @functools.partial(
        pl.kernel,
        mesh=mesh,
        compiler_params=pltpu.CompilerParams(needs_layout_passes=False),
        out_type=jax.ShapeDtypeStruct((e_total,), jnp.float32),
        scratch_types=[
            pltpu.VMEM((n_nodes,), jnp.float32),
            pltpu.VMEM((n_nodes,), jnp.float32),
            pltpu.VMEM((pad,), jnp.int32),
            pltpu.VMEM((pad,), jnp.int32),
            pltpu.VMEM((pad,), jnp.float32),
        ],
    )
    def k(p_hbm, q_hbm, i0_hbm, i1_hbm, out_hbm, p_v, q_v, i0_v, i1_v, out_v):
        wid = lax.axis_index("s") * nc + lax.axis_index("c")
        base = wid * ew
        pltpu.sync_copy(p_hbm, p_v)
        pltpu.sync_copy(q_hbm, q_v)
        if pad > ew:
            zeros16 = jnp.zeros((16,), jnp.int32)
            i0_v[pl.ds(pad - 16, 16)] = zeros16
            i1_v[pl.ds(pad - 16, 16)] = zeros16
        pltpu.sync_copy(i0_hbm.at[pl.ds(base, ew)], i0_v.at[pl.ds(0, ew)])
        pltpu.sync_copy(i1_hbm.at[pl.ds(base, ew)], i1_v.at[pl.ds(0, ew)])

        def body(k_it, _):
            off = k_it * 16
            g0 = plsc.load_gather(p_v, [i0_v[pl.ds(off, 16)]])
            g1 = plsc.load_gather(q_v, [i1_v[pl.ds(off, 16)]])
            out_v[pl.ds(off, 16)] = g0 + g1
            return _

        lax.fori_loop(0, steps, body, None)
        pltpu.sync_copy(out_v.at[pl.ds(0, ew)], out_hbm.at[pl.ds(base, ew)])

    return k(p, q, idx0, idx1)


def kernel(x, o_adj, s_adj, idx, Wo1, bo1, Wo2, bo2, Ws1, bs1, Ws2, bs2,
           ag1, ag2, Wd1, bd1, Wd2, bd2):
    n, nfeat = x.shape
    h1 = Wo1.shape[1]
    h2 = Wo2.shape[1]

    w = Wd1 @ Wd2
    wa = w[:h2, 0][None, :]
    wb = w[h2:, 0][None, :]
    c = (bd1 @ Wd2 + bd2).reshape(1, 1)

    bmp = 2000
    s_o, s_s = pl.pallas_call(
        _prep_body,
        grid=(pl.cdiv(n, bmp),),
        in_specs=[
            pl.BlockSpec((bmp, nfeat), lambda i: (i, 0)),
            pl.BlockSpec((nfeat, h1), lambda i: (0, 0)),
            pl.BlockSpec((nfeat, h1), lambda i: (0, 0)),
        ],
        out_specs=[
            pl.BlockSpec((bmp, h1), lambda i: (i, 0)),
            pl.BlockSpec((bmp, h1), lambda i: (i, 0)),
        ],
        out_shape=[
            jax.ShapeDtypeStruct((n, h1), jnp.float32),
            jax.ShapeDtypeStruct((n, h1), jnp.float32),
        ],
    )(x, Wo1, Ws1)

    nbr = n // _BM
    nbc = pl.cdiv(n, _BN)
    grid2d = (nbr, nbc)

    adj_ij = pl.BlockSpec((_BM, _BN), lambda i, j: (i, j))
    row_spec = lambda w_: pl.BlockSpec((_BM, w_), lambda i, j: (i, 0))
    res = lambda r, c_: pl.BlockSpec((r, c_), lambda i, j: (0, 0))

    def pass1(adj, s, b1, w2):
        return pl.pallas_call(
            functools.partial(_pass1_body, n),
            grid=grid2d,
            in_specs=[adj_ij, res(n, h1), res(1, h1), res(h1, h2)],
            out_specs=[row_spec(h2), row_spec(h2)],
            out_shape=[jax.ShapeDtypeStruct((n, h2), jnp.float32)] * 2,
            scratch_shapes=[
                pltpu.VMEM((_BM, h1), jnp.float32),
                pltpu.VMEM((_BM, h2), jnp.float32),
                pltpu.VMEM((n, h2), jnp.float32),
            ],
        )(adj, s, b1[None, :], w2)

    t_o, u1 = pass1(o_adj, s_o, bo1, Wo2)
    t_s, v1 = pass1(s_adj, s_s, bs1, Ws2)

    def adj_upper(i, j):
        return (i, jnp.minimum(jnp.maximum(j, (_BM * i) // _BN), nbc - 1))

    adj_up = pl.BlockSpec((_BM, _BN), adj_upper)

    u = pl.pallas_call(
        functools.partial(_pass2o_body, n),
        grid=grid2d,
        in_specs=[adj_up, res(n, h2), row_spec(h2), res(1, h2)],
        out_specs=row_spec(h2),
        out_shape=jax.ShapeDtypeStruct((n, h2), jnp.float32),
        scratch_shapes=[pltpu.VMEM((_BM, h2), jnp.float32)],
    )(o_adj, t_o, u1, bo2[None, :])

    pq = pl.pallas_call(
        functools.partial(_pass2s_body, n),
        grid=grid2d,
        in_specs=[
            adj_up, res(n, h2), row_spec(h2), row_spec(h2),
            res(1, h2), res(1, h2), res(1, h2),
            res(1, h2), res(1, h2), res(1, 1),
        ],
        out_specs=pl.BlockSpec((_BM, 8), lambda i, j: (i, 0)),
        out_shape=jax.ShapeDtypeStruct((n, 8), jnp.float32),
        scratch_shapes=[pltpu.VMEM((_BM, h2), jnp.float32)],
    )(s_adj, t_s, v1, u, bs2[None, :],
      ag1[None, :], ag2[None, :], wa, wb, c)

    p = pq[:, 0]
    q = pq[:, 1]

    info = plsc.get_sparse_core_info()
    nc, ns = info.num_cores, info.num_subcores
    out = _edge_gather(p, q, idx[0], idx[1], nc * ns, nc)
    return out[:, None]
